# baseline (speedup 1.0000x reference)
import functools
import jax
import jax.numpy as jnp
from jax import lax
from jax.experimental import pallas as pl
from jax.experimental.pallas import tpu as pltpu
from jax.experimental.pallas import tpu_sc as plsc

_NC = 2
_NS = 16
_NW = _NC * _NS
_L = 16
_D = 64
_C = 128
_NBUF = 4
_SCALE = 8.0


def _make_sc_lookup(nsl):
    assert nsl % _NBUF == 0
    mesh = plsc.VectorSubcoreMesh(
        core_axis_name="c", subcore_axis_name="s",
        num_cores=_NC, num_subcores=_NS)

    @functools.partial(
        pl.kernel,
        mesh=mesh,
        out_type=jax.ShapeDtypeStruct((200, 8, 32, 8, _C), jnp.float32),
        scratch_types=[
            pltpu.VMEM((nsl, _C), jnp.int32),
            pltpu.VMEM((_NBUF, _C, _C), jnp.float32),
            pltpu.VMEM((_NBUF, 8, 8, _C + 1), jnp.float32),
            pltpu.SemaphoreType.DMA,
            pltpu.SemaphoreType.DMA,
        ],
        compiler_params=pltpu.CompilerParams(
            use_tc_tiling_on_sc=False, needs_layout_passes=False),
    )
    def sc_lookup(x_hbm, table_hbm, out_hbm, idx_v, gbuf, sbuf, gsem, ssem):
        wid = lax.axis_index("s") * _NC + lax.axis_index("c")
        pltpu.sync_copy(x_hbm.at[wid], idx_v)
        iota = lax.iota(jnp.int32, _L)
        dvs = [iota + (cb * _L) for cb in range(_D // _L)]
        tdvs = [dv >> 3 for dv in dvs]
        rvs = [dv & 7 for dv in dvs]

        def slab_coords(m):
            g = wid * nsl + m
            j = (g >> 8) * 8 + (g & 7)
            ti = (g >> 3) & 31
            return j, ti

        for b in range(_NBUF):
            pltpu.async_copy(table_hbm.at[idx_v.at[b]], gbuf.at[b], gsem)

        def group(grp, carry):
            for b in range(_NBUF):
                m = grp * _NBUF + b
                pltpu.make_async_copy(
                    table_hbm.at[idx_v.at[0]], gbuf.at[b], gsem).wait()
                @pl.when(grp > 0)
                def _():
                    pltpu.make_async_copy(
                        sbuf.at[b, :, :, pl.ds(0, _C)],
                        out_hbm.at[0, :, 0], ssem).wait()

                @plsc.parallel_loop(0, _C, 1, unroll=4)
                def _(i):
                    iv = jnp.full((_L,), i, jnp.int32)
                    for cb in range(_D // _L):
                        v = gbuf[b, i, pl.ds(cb * _L, _L)] * _SCALE
                        plsc.store_scatter(
                            sbuf.at[b], [tdvs[cb], rvs[cb], iv], v)

                j, ti = slab_coords(m)
                pltpu.async_copy(
                    sbuf.at[b, :, :, pl.ds(0, _C)],
                    out_hbm.at[j, :, ti], ssem)
                @pl.when(m + _NBUF < nsl)
                def _():
                    pltpu.async_copy(
                        table_hbm.at[idx_v.at[m + _NBUF]], gbuf.at[b], gsem)
            return carry

        lax.fori_loop(0, nsl // _NBUF, group, 0)
        for b in range(_NBUF):
            pltpu.make_async_copy(
                sbuf.at[b, :, :, pl.ds(0, _C)],
                out_hbm.at[0, :, 0], ssem).wait()

    return sc_lookup


def kernel(x, table):
    bsz, seq = x.shape
    assert (bsz, seq) == (4096, 200)
    xp = (x.astype(jnp.int32).T
          .reshape(25, 8, 32, _C)
          .transpose(0, 2, 1, 3)
          .reshape(_NW, 200, _C))
    tpad = jnp.pad(table, ((0, 0), (0, _C - _D)))
    out_p = _make_sc_lookup(200)(xp, tpad)
    out = out_p.transpose(2, 4, 0, 1, 3).reshape(bsz, seq, _D)
    return out


# speedup vs baseline: 1.2888x; 1.2888x over previous
import functools
import jax
import jax.numpy as jnp
from jax import lax
from jax.experimental import pallas as pl
from jax.experimental.pallas import tpu as pltpu
from jax.experimental.pallas import tpu_sc as plsc

_NC = 2
_NS = 16
_NW = _NC * _NS
_L = 16
_D = 64
_C = 128
_NBUF = 4
_NREL = 4
_SCALE = 8.0
_V = 1000000
_NBLK = _V // _C
_BPW = _NBLK // _NW
_PITCH_A = 131
_PITCH_B = 129

_mesh = plsc.VectorSubcoreMesh(
    core_axis_name="c", subcore_axis_name="s",
    num_cores=_NC, num_subcores=_NS)


def _make_sc_relay():
    @functools.partial(
        pl.kernel,
        mesh=_mesh,
        out_type=jax.ShapeDtypeStruct((_NBLK * _D, _C), jnp.float32),
        scratch_types=[
            pltpu.VMEM((_NREL, _D, _C), jnp.float32),
            pltpu.SemaphoreType.DMA,
            pltpu.SemaphoreType.DMA,
        ],
        compiler_params=pltpu.CompilerParams(
            use_tc_tiling_on_sc=True, needs_layout_passes=False),
    )
    def relay(tt_hbm, out1_hbm, gbuf, isem, osem):
        wid = lax.axis_index("s") * _NC + lax.axis_index("c")

        def blk_of(t):
            return wid + t * _NW

        nread = _NREL // 2
        for b in range(nread):
            pltpu.async_copy(
                tt_hbm.at[:, pl.ds(blk_of(b) * _C, _C)], gbuf.at[b], isem)

        def group(grp, carry):
            for u in range(_NREL):
                t = grp * _NREL + u
                b = t % _NREL
                pltpu.make_async_copy(
                    tt_hbm.at[:, pl.ds(0, _C)], gbuf.at[b], isem).wait()
                pltpu.async_copy(
                    gbuf.at[b], out1_hbm.at[pl.ds(blk_of(t) * _D, _D)],
                    osem)
                @pl.when(t + nread < _BPW)
                def _():
                    @pl.when(t >= nread)
                    def _():
                        pltpu.make_async_copy(
                            gbuf.at[b], out1_hbm.at[pl.ds(0, _D)],
                            osem).wait()
                    pltpu.async_copy(
                        tt_hbm.at[:, pl.ds(blk_of(t + nread) * _C, _C)],
                        gbuf.at[(t + nread) % _NREL], isem)
            return carry

        lax.fori_loop(0, _BPW // _NREL, group, 0)
        for _t in range(_NBUF):
            pltpu.make_async_copy(
                gbuf.at[0], out1_hbm.at[pl.ds(0, _D)], osem).wait()

        @pl.when(wid < 4)
        def _():
            blk = _NBLK - 4 + wid
            pltpu.sync_copy(
                tt_hbm.at[:, pl.ds(blk * _C, _C)], gbuf.at[0])
            pltpu.sync_copy(
                gbuf.at[0], out1_hbm.at[pl.ds(blk * _D, _D)])

    return relay


def _make_sc_transpose():
    @functools.partial(
        pl.kernel,
        mesh=_mesh,
        out_type=jax.ShapeDtypeStruct((_V, _D), jnp.float32),
        scratch_types=[
            pltpu.VMEM((_NBUF, _D, _C), jnp.float32),
            pltpu.VMEM((_NBUF, _C, _PITCH_A), jnp.float32),
            pltpu.SemaphoreType.DMA,
            pltpu.SemaphoreType.DMA,
        ],
        compiler_params=pltpu.CompilerParams(
            use_tc_tiling_on_sc=False, needs_layout_passes=False),
    )
    def transpose(out1_hbm, tail_hbm, t2_hbm, gbuf, sbuf, isem, osem):
        wid = lax.axis_index("s") * _NC + lax.axis_index("c")
        iota = lax.iota(jnp.int32, _L)
        rvs = [iota + vb * _L for vb in range(_C // _L)]

        def blk_of(t):
            return wid + t * _NW

        def transpose_block(b):
            @plsc.parallel_loop(0, _D, 1, unroll=4)
            def _(d):
                dv = jnp.full((_L,), d, jnp.int32)
                for vb in range(_C // _L):
                    v = gbuf[b, d, pl.ds(vb * _L, _L)]
                    plsc.store_scatter(sbuf.at[b], [rvs[vb], dv], v)

        for b in range(_NBUF):
            pltpu.async_copy(
                out1_hbm.at[pl.ds(blk_of(b) * _D, _D)], gbuf.at[b], isem)

        def group(grp, carry):
            for b in range(_NBUF):
                t = grp * _NBUF + b
                pltpu.make_async_copy(
                    out1_hbm.at[pl.ds(0, _D)], gbuf.at[b], isem).wait()
                @pl.when(grp > 0)
                def _():
                    pltpu.make_async_copy(
                        sbuf.at[b, :, pl.ds(0, _D)],
                        t2_hbm.at[pl.ds(0, _C)], osem).wait()
                transpose_block(b)
                pltpu.async_copy(
                    sbuf.at[b, :, pl.ds(0, _D)],
                    t2_hbm.at[pl.ds(blk_of(t) * _C, _C)], osem)
                @pl.when(t + _NBUF < _BPW)
                def _():
                    pltpu.async_copy(
                        out1_hbm.at[pl.ds(blk_of(t + _NBUF) * _D, _D)],
                        gbuf.at[b], isem)
            return carry

        lax.fori_loop(0, _BPW // _NBUF, group, 0)
        for b in range(_NBUF):
            pltpu.make_async_copy(
                sbuf.at[b, :, pl.ds(0, _D)],
                t2_hbm.at[pl.ds(0, _C)], osem).wait()

        @pl.when(wid < 4)
        def _():
            blk = _NBLK - 4 + wid
            pltpu.sync_copy(
                out1_hbm.at[pl.ds(blk * _D, _D)], gbuf.at[0])
            transpose_block(0)
            pltpu.sync_copy(
                sbuf.at[0, :, pl.ds(0, _D)],
                t2_hbm.at[pl.ds(blk * _C, _C)])

        @pl.when(wid == 4)
        def _():
            pltpu.sync_copy(tail_hbm, gbuf.at[0, :, pl.ds(0, _D)])
            transpose_block(0)
            pltpu.sync_copy(
                sbuf.at[0, pl.ds(0, _D), pl.ds(0, _D)],
                t2_hbm.at[pl.ds(_NBLK * _C, _D)])

    return transpose


def _make_sc_lookup(nsl):
    assert nsl % _NBUF == 0

    @functools.partial(
        pl.kernel,
        mesh=_mesh,
        out_type=jax.ShapeDtypeStruct((200, 8, 32, 8, _C), jnp.float32),
        scratch_types=[
            pltpu.VMEM((nsl, _C), jnp.int32),
            pltpu.VMEM((_NBUF, _C, _D), jnp.float32),
            pltpu.VMEM((_NBUF, 8, 8, _PITCH_B), jnp.float32),
            pltpu.SemaphoreType.DMA,
            pltpu.SemaphoreType.DMA,
        ],
        compiler_params=pltpu.CompilerParams(
            use_tc_tiling_on_sc=False, needs_layout_passes=False),
    )
    def sc_lookup(x_hbm, t2_hbm, out_hbm, xv, gbuf, sbuf, gsem, ssem):
        wid = lax.axis_index("s") * _NC + lax.axis_index("c")
        pltpu.sync_copy(x_hbm.at[wid], xv)
        iota = lax.iota(jnp.int32, _L)
        dvs = [iota + (cb * _L) for cb in range(_D // _L)]
        tdvs = [dv >> 3 for dv in dvs]
        rvs = [dv & 7 for dv in dvs]

        def slab_coords(m):
            g = wid * nsl + m
            j = (g >> 8) * 8 + (g & 7)
            ti = (g >> 3) & 31
            return j, ti

        for b in range(_NBUF):
            pltpu.async_copy(t2_hbm.at[xv.at[b]], gbuf.at[b], gsem)

        def group(grp, carry):
            for b in range(_NBUF):
                m = grp * _NBUF + b
                pltpu.make_async_copy(
                    t2_hbm.at[xv.at[0]], gbuf.at[b], gsem).wait()
                @pl.when(grp > 0)
                def _():
                    pltpu.make_async_copy(
                        sbuf.at[b, :, :, pl.ds(0, _C)],
                        out_hbm.at[0, :, 0], ssem).wait()

                @plsc.parallel_loop(0, _C, 1, unroll=4)
                def _(i):
                    iv = jnp.full((_L,), i, jnp.int32)
                    for cb in range(_D // _L):
                        v = gbuf[b, i, pl.ds(cb * _L, _L)] * _SCALE
                        plsc.store_scatter(
                            sbuf.at[b], [tdvs[cb], rvs[cb], iv], v)

                j, ti = slab_coords(m)
                pltpu.async_copy(
                    sbuf.at[b, :, :, pl.ds(0, _C)],
                    out_hbm.at[j, :, ti], ssem)
                @pl.when(m + _NBUF < nsl)
                def _():
                    pltpu.async_copy(
                        t2_hbm.at[xv.at[m + _NBUF]], gbuf.at[b], gsem)
            return carry

        lax.fori_loop(0, nsl // _NBUF, group, 0)
        for b in range(_NBUF):
            pltpu.make_async_copy(
                sbuf.at[b, :, :, pl.ds(0, _C)],
                out_hbm.at[0, :, 0], ssem).wait()

    return sc_lookup


def kernel(x, table):
    bsz, seq = x.shape
    assert (bsz, seq) == (4096, 200)
    xp = (x.astype(jnp.int32).T
          .reshape(25, 8, 32, _C)
          .transpose(0, 2, 1, 3)
          .reshape(_NW, 200, _C))
    tail = table[_NBLK * _C:].T
    out1 = _make_sc_relay()(table.T)
    t2 = _make_sc_transpose()(out1, tail)
    out_p = _make_sc_lookup(200)(xp, t2)
    out = out_p.transpose(2, 4, 0, 1, 3).reshape(bsz, seq, _D)
    return out
